# baseline (speedup 1.0000x reference)
import functools

import jax
import jax.numpy as jnp
from jax import lax
from jax.experimental import pallas as pl
from jax.experimental.pallas import tpu as pltpu
from jax.experimental.pallas import tpu_sc as plsc

N_TOKENS = 2097152
N_CLASSES = 4
NUM_CORES = 2
NUM_SUBCORES = 16
NW = NUM_CORES * NUM_SUBCORES
TOK_PER_W = N_TOKENS // NW
CHUNK = 4096
N_CHUNKS = TOK_PER_W // CHUNK
LANES = 16

_LN2 = 0.6931471805599453
_SQRT2 = 1.4142135623730951


def _mesh():
    return plsc.VectorSubcoreMesh(core_axis_name="c", subcore_axis_name="s")


_CPARAMS = pltpu.CompilerParams(
    use_tc_tiling_on_sc=False, needs_layout_passes=False
)


def _worker_id():
    return lax.axis_index("s") * NUM_CORES + lax.axis_index("c")


def _neg_log(p):
    bits = plsc.bitcast(p, jnp.int32)
    e = (bits >> 23) - 127
    m = plsc.bitcast((bits & 0x007FFFFF) | 0x3F800000, jnp.float32)
    big = m > _SQRT2
    m = jnp.where(big, m * 0.5, m)
    ef = e.astype(jnp.float32) + jnp.where(big, 1.0, 0.0)
    z = (m - 1.0) / (m + 1.0)
    z2 = z * z
    poly = 1.0 + z2 * (1.0 / 3.0 + z2 * (1.0 / 5.0 + z2 * (1.0 / 7.0 + z2 * (1.0 / 9.0))))
    logm = 2.0 * z * poly
    return -(ef * _LN2 + logm)


def _gather_probs_and_labels(obuf, lbuf, j):
    lab = lbuf[pl.ds(j * LANES, LANES)]
    rows = lax.iota(jnp.int32, LANES) + j * LANES
    p = plsc.load_gather(obuf, [rows, lab])
    return p, lab


@functools.partial(
    pl.kernel,
    mesh=_mesh(),
    compiler_params=_CPARAMS,
    out_type=jax.ShapeDtypeStruct((NW, 48), jnp.float32),
    scratch_types=[
        pltpu.VMEM((CHUNK, N_CLASSES), jnp.float32),
        pltpu.VMEM((CHUNK,), jnp.int32),
        pltpu.VMEM((48,), jnp.float32),
    ],
)
def _phase1(out_hbm, lab_hbm, res_hbm, obuf, lbuf, rbuf):
    wid = _worker_id()
    base = wid * TOK_PER_W

    def chunk_body(ci, carry):
        s1, s2, cnt = carry
        tok0 = base + ci * CHUNK
        pltpu.sync_copy(out_hbm.at[pl.ds(tok0, CHUNK)], obuf)
        pltpu.sync_copy(lab_hbm.at[pl.ds(tok0, CHUNK)], lbuf)

        def vec_body(j, c2):
            s1, s2, cnt = c2
            p, lab = _gather_probs_and_labels(obuf, lbuf, j)
            cost = _neg_log(p)
            is_i = lab > 0
            s1 = s1 + jnp.where(is_i, cost, 0.0)
            s2 = s2 + jnp.where(is_i, 0.0, cost)
            cnt = cnt + jnp.where(is_i, 1.0, 0.0)
            return s1, s2, cnt

        return lax.fori_loop(0, CHUNK // LANES, vec_body, (s1, s2, cnt))

    zeros = jnp.zeros((LANES,), jnp.float32)
    s1, s2, cnt = lax.fori_loop(0, N_CHUNKS, chunk_body, (zeros, zeros, zeros))
    rbuf[pl.ds(0, LANES)] = s1
    rbuf[pl.ds(LANES, LANES)] = s2
    rbuf[pl.ds(2 * LANES, LANES)] = cnt
    pltpu.sync_copy(rbuf, res_hbm.at[wid])


def _make_digit_pass(digit_idx):
    shift = 24 - 8 * digit_idx

    @functools.partial(
        pl.kernel,
        mesh=_mesh(),
        compiler_params=_CPARAMS,
        out_type=jax.ShapeDtypeStruct((NW, 256 * LANES), jnp.int32),
        scratch_types=[
            pltpu.VMEM((CHUNK, N_CLASSES), jnp.float32),
            pltpu.VMEM((CHUNK,), jnp.int32),
            pltpu.VMEM((LANES,), jnp.int32),
            pltpu.VMEM((256 * LANES,), jnp.int32),
        ],
    )
    def _digit_pass(out_hbm, lab_hbm, pref_hbm, hist_hbm, obuf, lbuf, pbuf, hist):
        wid = _worker_id()
        base = wid * TOK_PER_W
        pltpu.sync_copy(pref_hbm, pbuf)
        prefix = pbuf[pl.ds(0, LANES)][0]
        zeros = jnp.zeros((LANES,), jnp.int32)

        def zero_body(i, _):
            hist[pl.ds(i * LANES, LANES)] = zeros
            return 0

        lax.fori_loop(0, 256, zero_body, 0)

        lane_iota = lax.iota(jnp.int32, LANES)
        ones = jnp.ones((LANES,), jnp.int32)

        def chunk_body(ci, _):
            tok0 = base + ci * CHUNK
            pltpu.sync_copy(out_hbm.at[pl.ds(tok0, CHUNK)], obuf)
            pltpu.sync_copy(lab_hbm.at[pl.ds(tok0, CHUNK)], lbuf)

            def vec_body(j, _2):
                p, lab = _gather_probs_and_labels(obuf, lbuf, j)
                bits = plsc.bitcast(p, jnp.int32)
                digit = (bits >> shift) & 0xFF
                mask = lab == 0
                if digit_idx > 0:
                    mask = mask & ((bits >> (shift + 8)) == prefix)
                plsc.addupdate_scatter(hist, [digit * LANES + lane_iota], ones, mask=mask)
                return 0

            return lax.fori_loop(0, CHUNK // LANES, vec_body, 0)

        lax.fori_loop(0, N_CHUNKS, chunk_body, 0)
        pltpu.sync_copy(hist, hist_hbm.at[wid])

    return _digit_pass


@functools.partial(
    pl.kernel,
    mesh=_mesh(),
    compiler_params=_CPARAMS,
    out_type=jax.ShapeDtypeStruct((NW, 48), jnp.float32),
    scratch_types=[
        pltpu.VMEM((CHUNK, N_CLASSES), jnp.float32),
        pltpu.VMEM((CHUNK,), jnp.int32),
        pltpu.VMEM((LANES,), jnp.int32),
        pltpu.VMEM((48,), jnp.float32),
    ],
)
def _below_pass(out_hbm, lab_hbm, thr_hbm, res_hbm, obuf, lbuf, tbuf, rbuf):
    wid = _worker_id()
    base = wid * TOK_PER_W
    pltpu.sync_copy(thr_hbm, tbuf)
    threshold = tbuf[pl.ds(0, LANES)][0]

    def chunk_body(ci, carry):
        bsum, bcnt = carry
        tok0 = base + ci * CHUNK
        pltpu.sync_copy(out_hbm.at[pl.ds(tok0, CHUNK)], obuf)
        pltpu.sync_copy(lab_hbm.at[pl.ds(tok0, CHUNK)], lbuf)

        def vec_body(j, c2):
            bsum, bcnt = c2
            p, lab = _gather_probs_and_labels(obuf, lbuf, j)
            bits = plsc.bitcast(p, jnp.int32)
            sel = (lab == 0) & (bits < threshold)
            cost = _neg_log(p)
            bsum = bsum + jnp.where(sel, cost, 0.0)
            bcnt = bcnt + jnp.where(sel, 1.0, 0.0)
            return bsum, bcnt

        return lax.fori_loop(0, CHUNK // LANES, vec_body, (bsum, bcnt))

    zeros = jnp.zeros((LANES,), jnp.float32)
    bsum, bcnt = lax.fori_loop(0, N_CHUNKS, chunk_body, (zeros, zeros))
    rbuf[pl.ds(0, LANES)] = bsum
    rbuf[pl.ds(LANES, LANES)] = bcnt
    rbuf[pl.ds(2 * LANES, LANES)] = jnp.zeros((LANES,), jnp.float32)
    pltpu.sync_copy(rbuf, res_hbm.at[wid])


_DIGIT_PASSES = [_make_digit_pass(d) for d in range(4)]


def _topk_fallback(outputs, labels, s1, k, den):
    k = k.astype(jnp.int32)
    prefix = jnp.zeros((LANES,), jnp.int32)
    rank = k

    for d in range(4):
        hist = _DIGIT_PASSES[d](outputs, labels, prefix)
        h = jnp.sum(hist.reshape(NW, 256, LANES), axis=(0, 2))
        cum = jnp.cumsum(h)
        bstar = jnp.argmax(cum >= rank).astype(jnp.int32)
        c_lt = cum[bstar] - h[bstar]
        rank = rank - c_lt
        prefix = prefix.at[0].set((prefix[0] << 8) | bstar)

    threshold_bits = prefix[0]
    below = _below_pass(outputs, labels, prefix.at[0].set(threshold_bits))
    below = below.reshape(NW, 3, LANES)
    below_sum = jnp.sum(below[:, 0, :])
    below_cnt = jnp.sum(below[:, 1, :])
    thr_cost = -jnp.log(lax.bitcast_convert_type(threshold_bits, jnp.float32))
    topk_sum = below_sum + (k.astype(jnp.float32) - below_cnt) * thr_cost
    return (s1 + topk_sum) / den


def kernel(outputs, labels):
    parts = _phase1(outputs, labels).reshape(NW, 3, LANES)
    s1 = jnp.sum(parts[:, 0, :])
    s2 = jnp.sum(parts[:, 1, :])
    n_f = jnp.sum(parts[:, 2, :])
    n_i = jnp.round(n_f).astype(jnp.int32)
    m_i = N_TOKENS - n_i
    k = jnp.minimum((5 * n_i) // 2, m_i)
    den = ((7 * n_i) // 2).astype(jnp.float32)
    return lax.cond(
        k >= m_i,
        lambda: (s1 + s2) / den,
        lambda: _topk_fallback(outputs, labels, s1, k, den),
    )


# speedup vs baseline: 3.3522x; 3.3522x over previous
import functools

import jax
import jax.numpy as jnp
from jax import lax
from jax.experimental import pallas as pl
from jax.experimental.pallas import tpu as pltpu
from jax.experimental.pallas import tpu_sc as plsc

N_TOKENS = 2097152
N_CLASSES = 4
NUM_CORES = 2
NUM_SUBCORES = 16
NW = NUM_CORES * NUM_SUBCORES
TOK_PER_W = N_TOKENS // NW
CHUNK = 4096
N_CHUNKS = TOK_PER_W // CHUNK
LANES = 16

_LN2 = 0.6931471805599453
_SQRT2 = 1.4142135623730951


def _mesh():
    return plsc.VectorSubcoreMesh(core_axis_name="c", subcore_axis_name="s")


_CPARAMS = pltpu.CompilerParams(
    use_tc_tiling_on_sc=False, needs_layout_passes=False
)


def _worker_id():
    return lax.axis_index("s") * NUM_CORES + lax.axis_index("c")


def _neg_log(p):
    bits = plsc.bitcast(p, jnp.int32)
    e = (bits >> 23) - 127
    m = plsc.bitcast((bits & 0x007FFFFF) | 0x3F800000, jnp.float32)
    big = m > _SQRT2
    m = jnp.where(big, m * 0.5, m)
    ef = e.astype(jnp.float32) + jnp.where(big, 1.0, 0.0)
    z = (m - 1.0) / (m + 1.0)
    z2 = z * z
    poly = 1.0 + z2 * (1.0 / 3.0 + z2 * (1.0 / 5.0 + z2 * (1.0 / 7.0 + z2 * (1.0 / 9.0))))
    logm = 2.0 * z * poly
    return -(ef * _LN2 + logm)


def _gather_probs_and_labels(obuf, lbuf, j):
    lab = lbuf[pl.ds(j * LANES, LANES)]
    rows = lax.iota(jnp.int32, LANES) + j * LANES
    p = plsc.load_gather(obuf, [lab * CHUNK + rows])
    return p, lab


def _copy_chunk(c_hbms, lab_hbm, obuf, lbuf, tok0):
    for c in range(N_CLASSES):
        pltpu.sync_copy(
            c_hbms[c].at[pl.ds(tok0, CHUNK)], obuf.at[pl.ds(c * CHUNK, CHUNK)]
        )
    pltpu.sync_copy(lab_hbm.at[pl.ds(tok0, CHUNK)], lbuf)


_FLAT_BLK = 8192


@functools.partial(
    pl.pallas_call,
    out_shape=[jax.ShapeDtypeStruct((N_TOKENS,), jnp.float32)] * N_CLASSES,
    grid=(N_TOKENS // _FLAT_BLK,),
    in_specs=[pl.BlockSpec((_FLAT_BLK, N_CLASSES), lambda i: (i, 0))],
    out_specs=[pl.BlockSpec((_FLAT_BLK,), lambda i: (i,))] * N_CLASSES,
)
def _split_cols_tc(x_ref, o0_ref, o1_ref, o2_ref, o3_ref):
    xt = x_ref[...].T
    o0_ref[...] = xt[0]
    o1_ref[...] = xt[1]
    o2_ref[...] = xt[2]
    o3_ref[...] = xt[3]


@functools.partial(
    pl.kernel,
    mesh=_mesh(),
    compiler_params=_CPARAMS,
    out_type=jax.ShapeDtypeStruct((NW, 48), jnp.float32),
    scratch_types=[
        pltpu.VMEM((CHUNK * N_CLASSES,), jnp.float32),
        pltpu.VMEM((CHUNK,), jnp.int32),
        pltpu.VMEM((48,), jnp.float32),
    ],
)
def _phase1(c0_hbm, c1_hbm, c2_hbm, c3_hbm, lab_hbm, res_hbm, obuf, lbuf, rbuf):
    wid = _worker_id()
    base = wid * TOK_PER_W
    c_hbms = (c0_hbm, c1_hbm, c2_hbm, c3_hbm)

    def chunk_body(ci, carry):
        s1, s2, cnt = carry
        tok0 = base + ci * CHUNK
        _copy_chunk(c_hbms, lab_hbm, obuf, lbuf, tok0)

        def vec_body(j, c2):
            s1, s2, cnt = c2
            p, lab = _gather_probs_and_labels(obuf, lbuf, j)
            cost = _neg_log(p)
            is_i = lab > 0
            s1 = s1 + jnp.where(is_i, cost, 0.0)
            s2 = s2 + jnp.where(is_i, 0.0, cost)
            cnt = cnt + jnp.where(is_i, 1.0, 0.0)
            return s1, s2, cnt

        return lax.fori_loop(0, CHUNK // LANES, vec_body, (s1, s2, cnt))

    zeros = jnp.zeros((LANES,), jnp.float32)
    s1, s2, cnt = lax.fori_loop(0, N_CHUNKS, chunk_body, (zeros, zeros, zeros))
    rbuf[pl.ds(0, LANES)] = s1
    rbuf[pl.ds(LANES, LANES)] = s2
    rbuf[pl.ds(2 * LANES, LANES)] = cnt
    pltpu.sync_copy(rbuf, res_hbm.at[wid])


def _make_digit_pass(digit_idx):
    shift = 24 - 8 * digit_idx

    @functools.partial(
        pl.kernel,
        mesh=_mesh(),
        compiler_params=_CPARAMS,
        out_type=jax.ShapeDtypeStruct((NW, 256 * LANES), jnp.int32),
        scratch_types=[
            pltpu.VMEM((CHUNK * N_CLASSES,), jnp.float32),
            pltpu.VMEM((CHUNK,), jnp.int32),
            pltpu.VMEM((LANES,), jnp.int32),
            pltpu.VMEM((256 * LANES,), jnp.int32),
        ],
    )
    def _digit_pass(c0_hbm, c1_hbm, c2_hbm, c3_hbm, lab_hbm, pref_hbm, hist_hbm,
                    obuf, lbuf, pbuf, hist):
        wid = _worker_id()
        base = wid * TOK_PER_W
        c_hbms = (c0_hbm, c1_hbm, c2_hbm, c3_hbm)
        pltpu.sync_copy(pref_hbm, pbuf)
        prefix = pbuf[pl.ds(0, LANES)][0]
        zeros = jnp.zeros((LANES,), jnp.int32)

        def zero_body(i, _):
            hist[pl.ds(i * LANES, LANES)] = zeros
            return 0

        lax.fori_loop(0, 256, zero_body, 0)

        lane_iota = lax.iota(jnp.int32, LANES)
        ones = jnp.ones((LANES,), jnp.int32)

        def chunk_body(ci, _):
            tok0 = base + ci * CHUNK
            _copy_chunk(c_hbms, lab_hbm, obuf, lbuf, tok0)

            def vec_body(j, _2):
                p, lab = _gather_probs_and_labels(obuf, lbuf, j)
                bits = plsc.bitcast(p, jnp.int32)
                digit = (bits >> shift) & 0xFF
                mask = lab == 0
                if digit_idx > 0:
                    mask = mask & ((bits >> (shift + 8)) == prefix)
                plsc.addupdate_scatter(hist, [digit * LANES + lane_iota], ones, mask=mask)
                return 0

            return lax.fori_loop(0, CHUNK // LANES, vec_body, 0)

        lax.fori_loop(0, N_CHUNKS, chunk_body, 0)
        pltpu.sync_copy(hist, hist_hbm.at[wid])

    return _digit_pass


@functools.partial(
    pl.kernel,
    mesh=_mesh(),
    compiler_params=_CPARAMS,
    out_type=jax.ShapeDtypeStruct((NW, 48), jnp.float32),
    scratch_types=[
        pltpu.VMEM((CHUNK * N_CLASSES,), jnp.float32),
        pltpu.VMEM((CHUNK,), jnp.int32),
        pltpu.VMEM((LANES,), jnp.int32),
        pltpu.VMEM((48,), jnp.float32),
    ],
)
def _below_pass(c0_hbm, c1_hbm, c2_hbm, c3_hbm, lab_hbm, thr_hbm, res_hbm,
                obuf, lbuf, tbuf, rbuf):
    wid = _worker_id()
    base = wid * TOK_PER_W
    c_hbms = (c0_hbm, c1_hbm, c2_hbm, c3_hbm)
    pltpu.sync_copy(thr_hbm, tbuf)
    threshold = tbuf[pl.ds(0, LANES)][0]

    def chunk_body(ci, carry):
        bsum, bcnt = carry
        tok0 = base + ci * CHUNK
        _copy_chunk(c_hbms, lab_hbm, obuf, lbuf, tok0)

        def vec_body(j, c2):
            bsum, bcnt = c2
            p, lab = _gather_probs_and_labels(obuf, lbuf, j)
            bits = plsc.bitcast(p, jnp.int32)
            sel = (lab == 0) & (bits < threshold)
            cost = _neg_log(p)
            bsum = bsum + jnp.where(sel, cost, 0.0)
            bcnt = bcnt + jnp.where(sel, 1.0, 0.0)
            return bsum, bcnt

        return lax.fori_loop(0, CHUNK // LANES, vec_body, (bsum, bcnt))

    zeros = jnp.zeros((LANES,), jnp.float32)
    bsum, bcnt = lax.fori_loop(0, N_CHUNKS, chunk_body, (zeros, zeros))
    rbuf[pl.ds(0, LANES)] = bsum
    rbuf[pl.ds(LANES, LANES)] = bcnt
    rbuf[pl.ds(2 * LANES, LANES)] = jnp.zeros((LANES,), jnp.float32)
    pltpu.sync_copy(rbuf, res_hbm.at[wid])


_DIGIT_PASSES = [_make_digit_pass(d) for d in range(4)]


def _topk_fallback(cols, labels, s1, k, den):
    k = k.astype(jnp.int32)
    prefix = jnp.zeros((LANES,), jnp.int32)
    rank = k

    for d in range(4):
        hist = _DIGIT_PASSES[d](*cols, labels, prefix)
        h = jnp.sum(hist.reshape(NW, 256, LANES), axis=(0, 2))
        cum = jnp.cumsum(h)
        bstar = jnp.argmax(cum >= rank).astype(jnp.int32)
        c_lt = cum[bstar] - h[bstar]
        rank = rank - c_lt
        prefix = prefix.at[0].set((prefix[0] << 8) | bstar)

    threshold_bits = prefix[0]
    below = _below_pass(*cols, labels, prefix.at[0].set(threshold_bits))
    below = below.reshape(NW, 3, LANES)
    below_sum = jnp.sum(below[:, 0, :])
    below_cnt = jnp.sum(below[:, 1, :])
    thr_cost = -jnp.log(lax.bitcast_convert_type(threshold_bits, jnp.float32))
    topk_sum = below_sum + (k.astype(jnp.float32) - below_cnt) * thr_cost
    return (s1 + topk_sum) / den


def kernel(outputs, labels):
    cols = _split_cols_tc(outputs)
    parts = _phase1(*cols, labels).reshape(NW, 3, LANES)
    s1 = jnp.sum(parts[:, 0, :])
    s2 = jnp.sum(parts[:, 1, :])
    n_f = jnp.sum(parts[:, 2, :])
    n_i = jnp.round(n_f).astype(jnp.int32)
    m_i = N_TOKENS - n_i
    k = jnp.minimum((5 * n_i) // 2, m_i)
    den = ((7 * n_i) // 2).astype(jnp.float32)
    return lax.cond(
        k >= m_i,
        lambda: (s1 + s2) / den,
        lambda: _topk_fallback(cols, labels, s1, k, den),
    )
